# baseline (speedup 1.0000x reference)
import functools

import jax
import jax.numpy as jnp
from jax import lax
from jax.experimental import pallas as pl
from jax.experimental.pallas import tpu as pltpu
from jax.experimental.pallas import tpu_sc as plsc

N = 4096
L = 16
NC = 2
NS = 16
NW = NC * NS
ROWS_PER_W = N // NW
RB = 4
NBUF = 2
NBLK = ROWS_PER_W // RB


def _flip_sc_body(x_hbm, out_hbm, in_v, out_v, in_s0, in_s1, out_s0, out_s1):
    wid = lax.axis_index("s") * NC + lax.axis_index("c")
    base = wid * ROWS_PER_W
    in_sems = (in_s0, in_s1)
    out_sems = (out_s0, out_s1)

    def in_copy(g, b):
        return pltpu.make_async_copy(
            x_hbm.at[pl.ds(base + g * RB, RB)], in_v.at[b], in_sems[b]
        )

    def out_copy(g, b):
        return pltpu.make_async_copy(
            out_v.at[b], out_hbm.at[pl.ds(base + g * RB, RB)], out_sems[b]
        )

    for b in range(NBUF):
        in_copy(b, b).start()

    for g in range(NBLK):
        b = g % NBUF
        in_copy(g, b).wait()
        if g >= NBUF:
            out_copy(g - NBUF, b).wait()

        @plsc.parallel_loop(0, N // L, step=1, unroll=16)
        def chunk(c):
            src = (N // L - 1 - c) * L
            for i in range(RB):
                v = in_v[b, i, pl.ds(src, L)]
                out_v[b, i, pl.ds(c * L, L)] = lax.rev(v, (0,))

        out_copy(g, b).start()
        if g + NBUF < NBLK:
            in_copy(g + NBUF, b).start()

    for g in range(NBLK - NBUF, NBLK):
        out_copy(g, g % NBUF).wait()


def kernel(inputs):
    flip = functools.partial(
        pl.kernel,
        mesh=plsc.VectorSubcoreMesh(core_axis_name="c", subcore_axis_name="s"),
        out_type=jax.ShapeDtypeStruct((N, N), jnp.float32),
        scratch_types=[
            pltpu.VMEM((NBUF, RB, N), jnp.float32),
            pltpu.VMEM((NBUF, RB, N), jnp.float32),
            pltpu.SemaphoreType.DMA,
            pltpu.SemaphoreType.DMA,
            pltpu.SemaphoreType.DMA,
            pltpu.SemaphoreType.DMA,
        ],
    )(_flip_sc_body)
    return flip(inputs)


# speedup vs baseline: 1.1319x; 1.1319x over previous
import functools

import jax
import jax.numpy as jnp
from jax import lax
from jax.experimental import pallas as pl
from jax.experimental.pallas import tpu as pltpu
from jax.experimental.pallas import tpu_sc as plsc

N = 4096
A = 2560

L = 16
NC = 2
NS = 16
NW = NC * NS
SC_ROWS = N - A
ROWS_PER_W = SC_ROWS // NW
RB = 4
NBUF = 2
NBLK = ROWS_PER_W // RB


def _flip_sc_body(x_hbm, out_hbm, in_v, out_v, in_s0, in_s1, out_s0, out_s1):
    wid = lax.axis_index("s") * NC + lax.axis_index("c")
    base = A + wid * ROWS_PER_W
    in_sems = (in_s0, in_s1)
    out_sems = (out_s0, out_s1)

    def in_copy(g, b):
        return pltpu.make_async_copy(
            x_hbm.at[pl.ds(base + g * RB, RB)], in_v.at[b], in_sems[b]
        )

    def out_copy(g, b):
        return pltpu.make_async_copy(
            out_v.at[b], out_hbm.at[pl.ds(base + g * RB, RB)], out_sems[b]
        )

    for b in range(NBUF):
        in_copy(b, b).start()

    for g in range(NBLK):
        b = g % NBUF
        in_copy(g, b).wait()
        if g >= NBUF:
            out_copy(g - NBUF, b).wait()

        @plsc.parallel_loop(0, N // L, step=1, unroll=8)
        def chunk(c):
            src = (N // L - 1 - c) * L
            for i in range(RB):
                v = in_v[b, i, pl.ds(src, L)]
                out_v[b, i, pl.ds(c * L, L)] = lax.rev(v, (0,))

        out_copy(g, b).start()
        if g + NBUF < NBLK:
            in_copy(g + NBUF, b).start()

    for g in range(NBLK - NBUF, NBLK):
        out_copy(g, g % NBUF).wait()


def _flip_sc(inputs):
    flip = functools.partial(
        pl.kernel,
        mesh=plsc.VectorSubcoreMesh(core_axis_name="c", subcore_axis_name="s"),
        out_type=jax.ShapeDtypeStruct((N, N), jnp.float32),
        scratch_types=[
            pltpu.VMEM((NBUF, RB, N), jnp.float32),
            pltpu.VMEM((NBUF, RB, N), jnp.float32),
            pltpu.SemaphoreType.DMA,
            pltpu.SemaphoreType.DMA,
            pltpu.SemaphoreType.DMA,
            pltpu.SemaphoreType.DMA,
        ],
    )(_flip_sc_body)
    return flip(inputs)


BLK_R = 512
STRIP = 128


def _flip_block_tc(x_ref, alias_ref, p_ref, o_ref):
    ns = N // STRIP
    p = p_ref[...]
    for s in range(ns):
        src = (ns - 1 - s) * STRIP
        o_ref[:, s * STRIP:(s + 1) * STRIP] = jax.lax.dot(
            x_ref[:, src:src + STRIP], p, preferred_element_type=jnp.float32
        )


def kernel(inputs):
    sc_out = _flip_sc(inputs)
    rev = jnp.equal(
        jnp.arange(STRIP)[:, None] + jnp.arange(STRIP)[None, :], STRIP - 1
    ).astype(jnp.float32)
    return pl.pallas_call(
        _flip_block_tc,
        grid=(A // BLK_R,),
        in_specs=[
            pl.BlockSpec((BLK_R, N), lambda i: (i, 0)),
            pl.BlockSpec(memory_space=pl.ANY),
            pl.BlockSpec((STRIP, STRIP), lambda i: (0, 0)),
        ],
        out_specs=pl.BlockSpec((BLK_R, N), lambda i: (i, 0)),
        out_shape=jax.ShapeDtypeStruct((N, N), jnp.float32),
        input_output_aliases={1: 0},
    )(inputs, sc_out, rev)


# speedup vs baseline: 1.1539x; 1.0195x over previous
import functools

import jax
import jax.numpy as jnp
from jax import lax
from jax.experimental import pallas as pl
from jax.experimental.pallas import tpu as pltpu
from jax.experimental.pallas import tpu_sc as plsc

N = 4096
A = 3072

L = 16
NC = 2
NS = 16
NW = NC * NS
SC_ROWS = N - A
ROWS_PER_W = SC_ROWS // NW
RB = 4
NBUF = 2
NBLK = ROWS_PER_W // RB


def _flip_sc_body(x_hbm, out_hbm, in_v, out_v, in_s0, in_s1, out_s0, out_s1):
    wid = lax.axis_index("s") * NC + lax.axis_index("c")
    base = A + wid * ROWS_PER_W
    in_sems = (in_s0, in_s1)
    out_sems = (out_s0, out_s1)

    def in_copy(g, b):
        return pltpu.make_async_copy(
            x_hbm.at[pl.ds(base + g * RB, RB)], in_v.at[b], in_sems[b]
        )

    def out_copy(g, b):
        return pltpu.make_async_copy(
            out_v.at[b], out_hbm.at[pl.ds(base + g * RB, RB)], out_sems[b]
        )

    for b in range(NBUF):
        in_copy(b, b).start()

    for g in range(NBLK):
        b = g % NBUF
        in_copy(g, b).wait()
        if g >= NBUF:
            out_copy(g - NBUF, b).wait()

        @plsc.parallel_loop(0, N // L, step=1, unroll=8)
        def chunk(c):
            src = (N // L - 1 - c) * L
            for i in range(RB):
                v = in_v[b, i, pl.ds(src, L)]
                out_v[b, i, pl.ds(c * L, L)] = lax.rev(v, (0,))

        out_copy(g, b).start()
        if g + NBUF < NBLK:
            in_copy(g + NBUF, b).start()

    for g in range(NBLK - NBUF, NBLK):
        out_copy(g, g % NBUF).wait()


def _flip_sc(inputs):
    flip = functools.partial(
        pl.kernel,
        mesh=plsc.VectorSubcoreMesh(core_axis_name="c", subcore_axis_name="s"),
        out_type=jax.ShapeDtypeStruct((N, N), jnp.float32),
        scratch_types=[
            pltpu.VMEM((NBUF, RB, N), jnp.float32),
            pltpu.VMEM((NBUF, RB, N), jnp.float32),
            pltpu.SemaphoreType.DMA,
            pltpu.SemaphoreType.DMA,
            pltpu.SemaphoreType.DMA,
            pltpu.SemaphoreType.DMA,
        ],
    )(_flip_sc_body)
    return flip(inputs)


BLK_R = 512
STRIP = 128


def _flip_block_tc(x_ref, alias_ref, p_ref, o_ref):
    ns = N // STRIP
    p = p_ref[...]
    for s in range(ns):
        src = (ns - 1 - s) * STRIP
        o_ref[:, s * STRIP:(s + 1) * STRIP] = jax.lax.dot(
            x_ref[:, src:src + STRIP], p, preferred_element_type=jnp.float32
        )


def kernel(inputs):
    sc_out = _flip_sc(inputs)
    rev = jnp.equal(
        jnp.arange(STRIP)[:, None] + jnp.arange(STRIP)[None, :], STRIP - 1
    ).astype(jnp.float32)
    return pl.pallas_call(
        _flip_block_tc,
        grid=(A // BLK_R,),
        in_specs=[
            pl.BlockSpec((BLK_R, N), lambda i: (i, 0)),
            pl.BlockSpec(memory_space=pl.ANY),
            pl.BlockSpec((STRIP, STRIP), lambda i: (0, 0)),
        ],
        out_specs=pl.BlockSpec((BLK_R, N), lambda i: (i, 0)),
        out_shape=jax.ShapeDtypeStruct((N, N), jnp.float32),
        input_output_aliases={1: 0},
    )(inputs, sc_out, rev)
